# baseline (speedup 1.0000x reference)
import jax
import jax.numpy as jnp
from jax import lax
from jax.experimental import pallas as pl
from jax.experimental.pallas import tpu as pltpu, tpu_sc as plsc

_N = 1_000_000
_C = 20
_M1 = 50_000
_M2 = 100_000

_P1P = 53_248
_CH1 = 1664
_W1 = 128
_NC1 = _CH1 // _W1

_P2P = 102_400
_CH2 = 3200
_W2 = 128
_NC2 = _CH2 // _W2

_SLOT_SZ = 32 * 31_360
_CODE_SZ = 32 * 31_360
_CM_REAL = _C * _M2
_CM_SZ = 32 * 64_192
_ZB = 7_840
_ZBC = 32_096


def _zero_fill(buf, n):
    zero = jnp.zeros((16,), buf.dtype)

    def body(i, _):
        buf[pl.ds(i * 16, 16)] = zero
        return 0

    lax.fori_loop(0, n // 16, body, 0)


def _k0_body(slot_hbm, cm_hbm, zslot, zcm):
    wid = lax.axis_index("s") * 2 + lax.axis_index("c")
    negone = jnp.full((16,), -1, jnp.int32)

    def nbody(i, _):
        zslot[pl.ds(i * 16, 16)] = negone
        return 0

    lax.fori_loop(0, _ZB // 16, nbody, 0)
    _zero_fill(zcm, _ZBC)
    sbase = wid * (4 * _ZB)
    for k in range(4):
        pltpu.sync_copy(zslot, slot_hbm.at[pl.ds(sbase + k * _ZB, _ZB)])
    cmbase = wid * (2 * _ZBC)
    for k in range(2):
        pltpu.sync_copy(zcm, cm_hbm.at[pl.ds(cmbase + k * _ZBC, _ZBC)])


def _load_2d(hbm, base, flat, ref2d, nc, w, sem):
    pltpu.sync_copy(hbm.at[pl.ds(base, nc * w)], flat)
    for cc in range(nc):
        def body(k, _, cc=cc):
            ref2d[cc, pl.ds(k * 16, 16)] = flat[pl.ds(cc * w + k * 16, 16)]
            return 0

        lax.fori_loop(0, w // 16, body, 0)


def _k1_body(p2_hbm, slot_hbm, dummy_hbm, p2f, p2v, jvals, sem):
    wid = lax.axis_index("s") * 2 + lax.axis_index("c")
    base = wid * _CH2
    _load_2d(p2_hbm, base, p2f, p2v, _NC2, _W2, sem)
    iota = lax.iota(jnp.int32, 16)

    def jbody(i, _):
        jvals[pl.ds(i * 16, 16)] = base + i * 16 + iota
        return 0

    lax.fori_loop(0, _CH2 // 16, jbody, 0)
    scats = [
        pltpu.async_copy(
            jvals.at[pl.ds(cc * _W2, _W2)], slot_hbm.at[p2v.at[cc]], sem
        )
        for cc in range(_NC2)
    ]
    for d in scats:
        d.wait()

    @pl.when(wid == 0)
    def _():
        pltpu.sync_copy(jvals.at[pl.ds(0, 16)], dummy_hbm)


def _k2_body(slot_hbm, p1_hbm, v1_hbm, cm_hbm, code_hbm,
             zbuf, p1v, v1v, jjv, cmi, ones, sem):
    wid = lax.axis_index("s") * 2 + lax.axis_index("c")
    _zero_fill(zbuf, _ZB)
    cbase = wid * (4 * _ZB)
    for k in range(4):
        pltpu.sync_copy(zbuf, code_hbm.at[pl.ds(cbase + k * _ZB, _ZB)])
    onev = jnp.full((16,), 1.0, jnp.float32)

    def obody(i, _):
        ones[pl.ds(i * 16, 16)] = onev
        return 0

    lax.fori_loop(0, _W1 // 16, obody, 0)
    base = wid * _CH1
    pltpu.sync_copy(p1_hbm.at[pl.ds(base, _CH1)], p1v)
    pltpu.sync_copy(v1_hbm.at[pl.ds(base, _CH1)], v1v)
    g1 = [
        pltpu.async_copy(
            slot_hbm.at[p1v.at[pl.ds(cc * _W1, _W1)]],
            jjv.at[pl.ds(cc * _W1, _W1)],
            sem,
        )
        for cc in range(_NC1)
    ]
    for d in g1:
        d.wait()

    iota = lax.iota(jnp.int32, 16)
    for cc in range(_NC1):
        def ibody(k, _, cc=cc):
            s = cc * _W1 + k * 16
            jj = jjv[pl.ds(s, 16)]
            v1 = v1v[pl.ds(s, 16)]
            gidx = base + s + iota
            valid = (jj >= 0) & (jj < _M2) & (gidx < _M1)
            jjc = jnp.clip(jj, 0, _M2 - 1)
            dump = _CM_REAL + gidx
            cmi[cc, pl.ds(k * 16, 16)] = jnp.where(valid, v1 * _M2 + jjc, dump)
            return 0

        lax.fori_loop(0, _W1 // 16, ibody, 0)
    scats = [
        pltpu.async_copy(ones, cm_hbm.at[cmi.at[cc]], sem)
        for cc in range(_NC1)
    ]
    for d in scats:
        d.wait()


def _k4_body(slot_hbm, p2_hbm, tcode_hbm, code_hbm, dummy_hbm,
             p2f, p2v, jjv, jjcv, tv, sem):
    wid = lax.axis_index("s") * 2 + lax.axis_index("c")
    base = wid * _CH2
    _load_2d(p2_hbm, base, p2f, p2v, _NC2, _W2, sem)
    g1 = [
        pltpu.async_copy(
            slot_hbm.at[p2v.at[cc]], jjv.at[pl.ds(cc * _W2, _W2)], sem
        )
        for cc in range(_NC2)
    ]
    for d in g1:
        d.wait()

    def cbody(i, _):
        jj = jjv[pl.ds(i * 16, 16)]
        jjcv[pl.ds(i * 16, 16)] = jnp.clip(jj, 0, _M2 - 1)
        return 0

    lax.fori_loop(0, _CH2 // 16, cbody, 0)
    g2 = [
        pltpu.async_copy(
            tcode_hbm.at[jjcv.at[pl.ds(cc * _W2, _W2)]],
            tv.at[pl.ds(cc * _W2, _W2)],
            sem,
        )
        for cc in range(_NC2)
    ]
    for d in g2:
        d.wait()
    scats = [
        pltpu.async_copy(
            tv.at[pl.ds(cc * _W2, _W2)], code_hbm.at[p2v.at[cc]], sem
        )
        for cc in range(_NC2)
    ]
    for d in scats:
        d.wait()

    @pl.when(wid == 0)
    def _():
        pltpu.sync_copy(jjcv.at[pl.ds(0, 16)], dummy_hbm)


def _k3_tc(cm2, WT, b_col):
    BJ = 12_288
    G = -(-_M2 // BJ)

    def body(cm_ref, wt_ref, b_ref, o_ref):
        cmb = cm_ref[...]
        touched = jnp.max(cmb, axis=0, keepdims=True) > 0.0
        logits = jnp.dot(
            wt_ref[...], cmb, preferred_element_type=jnp.float32
        ) + b_ref[...]
        m = jnp.max(logits, axis=0, keepdims=True)
        iota_c = lax.broadcasted_iota(jnp.int32, (_C, BJ), 0)
        sidx = jnp.min(
            jnp.where(logits == m, iota_c, _C), axis=0, keepdims=True
        )
        tc = jnp.where(touched, sidx + 2, 1)
        o_ref[...] = tc[0]

    return pl.pallas_call(
        body,
        out_shape=jax.ShapeDtypeStruct((_M2,), jnp.int32),
        grid=(G,),
        in_specs=[
            pl.BlockSpec((_C, BJ), lambda i: (0, i)),
            pl.BlockSpec((_C, _C), lambda i: (0, 0)),
            pl.BlockSpec((_C, 1), lambda i: (0, 0)),
        ],
        out_specs=pl.BlockSpec((BJ,), lambda i: (i,)),
    )(cm2, WT, b_col)


def _k5_tc(primaryT, code, W, b_row):
    BN = 32_768
    G = -(-_N // BN)

    def body(pm_ref, code_ref, w_ref, b_ref, o_ref):
        pm = pm_ref[...]
        codeb = code_ref[...][None, :]
        iota_c = lax.broadcasted_iota(jnp.int32, (_C, BN), 0)
        m = jnp.max(pm, axis=0, keepdims=True)
        k_n = jnp.min(
            jnp.where(pm == m, iota_c, _C), axis=0, keepdims=True
        )
        wb = w_ref[...] + b_ref[...]
        wm = jnp.max(wb, axis=1, keepdims=True)
        iota_cc = lax.broadcasted_iota(jnp.int32, (_C, _C), 1)
        ttab = jnp.min(
            jnp.where(wb == wm, iota_cc, _C), axis=1, keepdims=True
        )
        oh = (iota_c == k_n).astype(jnp.int32)
        tval = jnp.sum(oh * ttab, axis=0, keepdims=True)
        w_n = jnp.where(codeb >= 2, codeb - 2, tval)
        onehot = jnp.where(iota_c == w_n, 1.0, 0.0)
        o_ref[...] = jnp.where(codeb >= 1, onehot, pm)

    return pl.pallas_call(
        body,
        out_shape=jax.ShapeDtypeStruct((_C, _N), jnp.float32),
        grid=(G,),
        in_specs=[
            pl.BlockSpec((_C, BN), lambda i: (0, i)),
            pl.BlockSpec((BN,), lambda i: (i,)),
            pl.BlockSpec((_C, _C), lambda i: (0, 0)),
            pl.BlockSpec((1, _C), lambda i: (0, 0)),
        ],
        out_specs=pl.BlockSpec((_C, BN), lambda i: (0, i)),
    )(primaryT, code, W, b_row)


def kernel(primary, W, b, positions1, values1, positions2):
    p1 = positions1.astype(jnp.int32)
    v1 = values1.astype(jnp.int32)
    p2 = positions2.astype(jnp.int32)
    p1p = jnp.concatenate([p1, jnp.zeros((_P1P - _M1,), jnp.int32)])
    v1p = jnp.concatenate([v1, jnp.zeros((_P1P - _M1,), jnp.int32)])
    p2p = jnp.concatenate(
        [p2, _N + jnp.arange(_P2P - _M2, dtype=jnp.int32)]
    )

    mesh = plsc.VectorSubcoreMesh(core_axis_name="c", subcore_axis_name="s")

    k0 = pl.kernel(
        _k0_body,
        out_type=[
            jax.ShapeDtypeStruct((_SLOT_SZ,), jnp.int32),
            jax.ShapeDtypeStruct((_CM_SZ,), jnp.float32),
        ],
        mesh=mesh,
        scratch_types=[
            pltpu.VMEM((_ZB,), jnp.int32),
            pltpu.VMEM((_ZBC,), jnp.float32),
        ],
    )
    slot0, cm0 = k0()

    k1 = pl.kernel(
        _k1_body,
        out_type=jax.ShapeDtypeStruct((16,), jnp.int32),
        mesh=mesh,
        scratch_types=[
            pltpu.VMEM((_CH2,), jnp.int32),
            pltpu.VMEM((_NC2, _W2), jnp.int32),
            pltpu.VMEM((_CH2,), jnp.int32),
            pltpu.SemaphoreType.DMA,
        ],
    )
    slot_ref = jax.new_ref(slot0)
    k1(p2p, slot_ref)
    slot = slot_ref[...]

    k2 = pl.kernel(
        _k2_body,
        out_type=jax.ShapeDtypeStruct((_CODE_SZ,), jnp.int32),
        mesh=mesh,
        scratch_types=[
            pltpu.VMEM((_ZB,), jnp.int32),
            pltpu.VMEM((_CH1,), jnp.int32),
            pltpu.VMEM((_CH1,), jnp.int32),
            pltpu.VMEM((_CH1,), jnp.int32),
            pltpu.VMEM((_NC1, _W1), jnp.int32),
            pltpu.VMEM((_W1,), jnp.float32),
            pltpu.SemaphoreType.DMA,
        ],
    )
    cm_ref = jax.new_ref(cm0)
    code0 = k2(slot, p1p, v1p, cm_ref)
    cm = cm_ref[...]

    cm2 = cm[:_CM_REAL].reshape(_C, _M2)
    WT = jnp.swapaxes(W, 0, 1)
    b_col = b.reshape(_C, 1)
    tcode = _k3_tc(cm2, WT, b_col)

    k4 = pl.kernel(
        _k4_body,
        out_type=jax.ShapeDtypeStruct((16,), jnp.int32),
        mesh=mesh,
        scratch_types=[
            pltpu.VMEM((_CH2,), jnp.int32),
            pltpu.VMEM((_NC2, _W2), jnp.int32),
            pltpu.VMEM((_CH2,), jnp.int32),
            pltpu.VMEM((_CH2,), jnp.int32),
            pltpu.VMEM((_CH2,), jnp.int32),
            pltpu.SemaphoreType.DMA,
        ],
    )
    code_ref = jax.new_ref(code0)
    k4(slot, p2p, tcode, code_ref)
    code = code_ref[...]

    primaryT = jnp.swapaxes(primary, 0, 1)
    b_row = b.reshape(1, _C)
    outT = _k5_tc(primaryT, code[:_N], W, b_row)
    return jnp.swapaxes(outT, 0, 1)


# speedup vs baseline: 1.0629x; 1.0629x over previous
import jax
import jax.numpy as jnp
from jax import lax
from jax.experimental import pallas as pl
from jax.experimental.pallas import tpu as pltpu, tpu_sc as plsc

_N = 1_000_000
_C = 20
_M1 = 50_000
_M2 = 100_000

_P1P = 50_176
_CH1 = 1568
_W1 = 112
_NC1 = _CH1 // _W1

_P2P = 100_352
_CH2 = 3136
_W2 = 112
_NC2 = _CH2 // _W2

_SLOT_SZ = 32 * 31_360
_CODE_SZ = 32 * 31_360
_CM_REAL = _C * _M2
_CM_SZ = 32 * 64_128
_ZB = 7_840
_ZBC = 8_016


def _zero_fill(buf, n):
    zero = jnp.zeros((16,), buf.dtype)

    def body(i, _):
        buf[pl.ds(i * 16, 16)] = zero
        return 0

    lax.fori_loop(0, n // 16, body, 0)


def _k0_body(slot_hbm, cm_hbm, zslot, zcm):
    wid = lax.axis_index("s") * 2 + lax.axis_index("c")
    negone = jnp.full((16,), -1, jnp.int32)

    def nbody(i, _):
        zslot[pl.ds(i * 16, 16)] = negone
        return 0

    lax.fori_loop(0, _ZB // 16, nbody, 0)
    _zero_fill(zcm, _ZBC)
    sbase = wid * (4 * _ZB)
    for k in range(4):
        pltpu.sync_copy(zslot, slot_hbm.at[pl.ds(sbase + k * _ZB, _ZB)])
    cmbase = wid * (8 * _ZBC)
    for k in range(8):
        pltpu.sync_copy(zcm, cm_hbm.at[pl.ds(cmbase + k * _ZBC, _ZBC)])


def _load_2d(hbm, base, flat, ref2d, nc, w, sem):
    pltpu.sync_copy(hbm.at[pl.ds(base, nc * w)], flat)
    for cc in range(nc):
        def body(k, _, cc=cc):
            ref2d[cc, pl.ds(k * 16, 16)] = flat[pl.ds(cc * w + k * 16, 16)]
            return 0

        lax.fori_loop(0, w // 16, body, 0)


def _k1_body(p2_hbm, slot_hbm, dummy_hbm, p2f, p2v, jvals, sem):
    wid = lax.axis_index("s") * 2 + lax.axis_index("c")
    base = wid * _CH2
    _load_2d(p2_hbm, base, p2f, p2v, _NC2, _W2, sem)
    iota = lax.iota(jnp.int32, 16)

    def jbody(i, _):
        jvals[pl.ds(i * 16, 16)] = base + i * 16 + iota
        return 0

    lax.fori_loop(0, _CH2 // 16, jbody, 0)
    scats = [
        pltpu.async_copy(
            jvals.at[pl.ds(cc * _W2, _W2)], slot_hbm.at[p2v.at[cc]], sem
        )
        for cc in range(_NC2)
    ]
    for d in scats:
        d.wait()

    @pl.when(wid == 0)
    def _():
        pltpu.sync_copy(jvals.at[pl.ds(0, 16)], dummy_hbm)


def _k2_body(slot_hbm, p1_hbm, v1_hbm, cm_hbm, code_hbm,
             zbuf, p1v, v1v, jjv, cmi, ones, sem):
    wid = lax.axis_index("s") * 2 + lax.axis_index("c")
    _zero_fill(zbuf, _ZB)
    cbase = wid * (4 * _ZB)
    for k in range(4):
        pltpu.sync_copy(zbuf, code_hbm.at[pl.ds(cbase + k * _ZB, _ZB)])
    onev = jnp.full((16,), 1.0, jnp.float32)

    def obody(i, _):
        ones[pl.ds(i * 16, 16)] = onev
        return 0

    lax.fori_loop(0, _W1 // 16, obody, 0)
    base = wid * _CH1
    pltpu.sync_copy(p1_hbm.at[pl.ds(base, _CH1)], p1v)
    pltpu.sync_copy(v1_hbm.at[pl.ds(base, _CH1)], v1v)
    g1 = [
        pltpu.async_copy(
            slot_hbm.at[p1v.at[pl.ds(cc * _W1, _W1)]],
            jjv.at[pl.ds(cc * _W1, _W1)],
            sem,
        )
        for cc in range(_NC1)
    ]
    for d in g1:
        d.wait()

    iota = lax.iota(jnp.int32, 16)
    for cc in range(_NC1):
        def ibody(k, _, cc=cc):
            s = cc * _W1 + k * 16
            jj = jjv[pl.ds(s, 16)]
            v1 = v1v[pl.ds(s, 16)]
            gidx = base + s + iota
            valid = (jj >= 0) & (jj < _M2) & (gidx < _M1)
            jjc = jnp.clip(jj, 0, _M2 - 1)
            dump = _CM_REAL + gidx
            cmi[cc, pl.ds(k * 16, 16)] = jnp.where(valid, v1 * _M2 + jjc, dump)
            return 0

        lax.fori_loop(0, _W1 // 16, ibody, 0)
    scats = [
        pltpu.async_copy(ones, cm_hbm.at[cmi.at[cc]], sem)
        for cc in range(_NC1)
    ]
    for d in scats:
        d.wait()


def _k4_body(slot_hbm, p2_hbm, tcode_hbm, code_hbm, dummy_hbm,
             p2f, p2v, jjv, jjcv, tv, sem):
    wid = lax.axis_index("s") * 2 + lax.axis_index("c")
    base = wid * _CH2
    _load_2d(p2_hbm, base, p2f, p2v, _NC2, _W2, sem)
    g1 = [
        pltpu.async_copy(
            slot_hbm.at[p2v.at[cc]], jjv.at[pl.ds(cc * _W2, _W2)], sem
        )
        for cc in range(_NC2)
    ]
    for d in g1:
        d.wait()

    def cbody(i, _):
        jj = jjv[pl.ds(i * 16, 16)]
        jjcv[pl.ds(i * 16, 16)] = jnp.clip(jj, 0, _M2 - 1)
        return 0

    lax.fori_loop(0, _CH2 // 16, cbody, 0)
    g2 = [
        pltpu.async_copy(
            tcode_hbm.at[jjcv.at[pl.ds(cc * _W2, _W2)]],
            tv.at[pl.ds(cc * _W2, _W2)],
            sem,
        )
        for cc in range(_NC2)
    ]
    for d in g2:
        d.wait()
    scats = [
        pltpu.async_copy(
            tv.at[pl.ds(cc * _W2, _W2)], code_hbm.at[p2v.at[cc]], sem
        )
        for cc in range(_NC2)
    ]
    for d in scats:
        d.wait()

    @pl.when(wid == 0)
    def _():
        pltpu.sync_copy(jjcv.at[pl.ds(0, 16)], dummy_hbm)


def _k3_tc(cm2, WT, b_col):
    BJ = 12_288
    G = -(-_M2 // BJ)

    def body(cm_ref, wt_ref, b_ref, o_ref):
        cmb = cm_ref[...]
        touched = jnp.max(cmb, axis=0, keepdims=True) > 0.0
        logits = jnp.dot(
            wt_ref[...], cmb, preferred_element_type=jnp.float32
        ) + b_ref[...]
        m = jnp.max(logits, axis=0, keepdims=True)
        iota_c = lax.broadcasted_iota(jnp.int32, (_C, BJ), 0)
        sidx = jnp.min(
            jnp.where(logits == m, iota_c, _C), axis=0, keepdims=True
        )
        tc = jnp.where(touched, sidx + 2, 1)
        o_ref[...] = tc[0]

    return pl.pallas_call(
        body,
        out_shape=jax.ShapeDtypeStruct((_M2,), jnp.int32),
        grid=(G,),
        in_specs=[
            pl.BlockSpec((_C, BJ), lambda i: (0, i)),
            pl.BlockSpec((_C, _C), lambda i: (0, 0)),
            pl.BlockSpec((_C, 1), lambda i: (0, 0)),
        ],
        out_specs=pl.BlockSpec((BJ,), lambda i: (i,)),
    )(cm2, WT, b_col)


def _k5_tc(primaryT, code, W, b_row):
    BN = 32_768
    G = -(-_N // BN)

    def body(pm_ref, code_ref, w_ref, b_ref, o_ref):
        pm = pm_ref[...]
        codeb = code_ref[...][None, :]
        iota_c = lax.broadcasted_iota(jnp.int32, (_C, BN), 0)
        m = jnp.max(pm, axis=0, keepdims=True)
        k_n = jnp.min(
            jnp.where(pm == m, iota_c, _C), axis=0, keepdims=True
        )
        wb = w_ref[...] + b_ref[...]
        wm = jnp.max(wb, axis=1, keepdims=True)
        iota_cc = lax.broadcasted_iota(jnp.int32, (_C, _C), 1)
        ttab = jnp.min(
            jnp.where(wb == wm, iota_cc, _C), axis=1, keepdims=True
        )
        oh = (iota_c == k_n).astype(jnp.int32)
        tval = jnp.sum(oh * ttab, axis=0, keepdims=True)
        w_n = jnp.where(codeb >= 2, codeb - 2, tval)
        onehot = jnp.where(iota_c == w_n, 1.0, 0.0)
        o_ref[...] = jnp.where(codeb >= 1, onehot, pm)

    return pl.pallas_call(
        body,
        out_shape=jax.ShapeDtypeStruct((_C, _N), jnp.float32),
        grid=(G,),
        in_specs=[
            pl.BlockSpec((_C, BN), lambda i: (0, i)),
            pl.BlockSpec((BN,), lambda i: (i,)),
            pl.BlockSpec((_C, _C), lambda i: (0, 0)),
            pl.BlockSpec((1, _C), lambda i: (0, 0)),
        ],
        out_specs=pl.BlockSpec((_C, BN), lambda i: (0, i)),
    )(primaryT, code, W, b_row)


def kernel(primary, W, b, positions1, values1, positions2):
    p1 = positions1.astype(jnp.int32)
    v1 = values1.astype(jnp.int32)
    p2 = positions2.astype(jnp.int32)
    p1p = jnp.concatenate([p1, jnp.zeros((_P1P - _M1,), jnp.int32)])
    v1p = jnp.concatenate([v1, jnp.zeros((_P1P - _M1,), jnp.int32)])
    p2p = jnp.concatenate(
        [p2, _N + jnp.arange(_P2P - _M2, dtype=jnp.int32)]
    )

    mesh = plsc.VectorSubcoreMesh(core_axis_name="c", subcore_axis_name="s")

    k0 = pl.kernel(
        _k0_body,
        out_type=[
            jax.ShapeDtypeStruct((_SLOT_SZ,), jnp.int32),
            jax.ShapeDtypeStruct((_CM_SZ,), jnp.float32),
        ],
        mesh=mesh,
        scratch_types=[
            pltpu.VMEM((_ZB,), jnp.int32),
            pltpu.VMEM((_ZBC,), jnp.float32),
        ],
    )
    slot0, cm0 = k0()

    k1 = pl.kernel(
        _k1_body,
        out_type=jax.ShapeDtypeStruct((16,), jnp.int32),
        mesh=mesh,
        scratch_types=[
            pltpu.VMEM((_CH2,), jnp.int32),
            pltpu.VMEM((_NC2, _W2), jnp.int32),
            pltpu.VMEM((_CH2,), jnp.int32),
            pltpu.SemaphoreType.DMA,
        ],
    )
    slot_ref = jax.new_ref(slot0)
    k1(p2p, slot_ref)
    slot = slot_ref[...]

    k2 = pl.kernel(
        _k2_body,
        out_type=jax.ShapeDtypeStruct((_CODE_SZ,), jnp.int32),
        mesh=mesh,
        scratch_types=[
            pltpu.VMEM((_ZB,), jnp.int32),
            pltpu.VMEM((_CH1,), jnp.int32),
            pltpu.VMEM((_CH1,), jnp.int32),
            pltpu.VMEM((_CH1,), jnp.int32),
            pltpu.VMEM((_NC1, _W1), jnp.int32),
            pltpu.VMEM((_W1,), jnp.float32),
            pltpu.SemaphoreType.DMA,
        ],
    )
    cm_ref = jax.new_ref(cm0)
    code0 = k2(slot, p1p, v1p, cm_ref)
    cm = cm_ref[...]

    cm2 = cm[:_CM_REAL].reshape(_C, _M2)
    WT = jnp.swapaxes(W, 0, 1)
    b_col = b.reshape(_C, 1)
    tcode = _k3_tc(cm2, WT, b_col)

    k4 = pl.kernel(
        _k4_body,
        out_type=jax.ShapeDtypeStruct((16,), jnp.int32),
        mesh=mesh,
        scratch_types=[
            pltpu.VMEM((_CH2,), jnp.int32),
            pltpu.VMEM((_NC2, _W2), jnp.int32),
            pltpu.VMEM((_CH2,), jnp.int32),
            pltpu.VMEM((_CH2,), jnp.int32),
            pltpu.VMEM((_CH2,), jnp.int32),
            pltpu.SemaphoreType.DMA,
        ],
    )
    code_ref = jax.new_ref(code0)
    k4(slot, p2p, tcode, code_ref)
    code = code_ref[...]

    primaryT = jnp.swapaxes(primary, 0, 1)
    b_row = b.reshape(1, _C)
    outT = _k5_tc(primaryT, code[:_N], W, b_row)
    return jnp.swapaxes(outT, 0, 1)


# speedup vs baseline: 1.2922x; 1.2158x over previous
import jax
import jax.numpy as jnp
from jax import lax
from jax.experimental import pallas as pl
from jax.experimental.pallas import tpu as pltpu, tpu_sc as plsc

_N = 1_000_000
_C = 20
_M1 = 50_000
_M2 = 100_000

_P1P = 50_176
_CH1 = 1568
_W1 = 112
_NC1 = _CH1 // _W1

_P2P = 100_352
_CH2 = 3136
_W2 = 112
_NC2 = _CH2 // _W2

_SLOT_SZ = 32 * 31_360
_CODE_SZ = 32 * 31_360
_CM_REAL = _C * _M2
_CM_SZ = 32 * 64_128
_ZB = 7_840
_ZBC = 8_016


def _zero_fill(buf, n):
    zero = jnp.zeros((16,), buf.dtype)

    def body(i, _):
        buf[pl.ds(i * 16, 16)] = zero
        return 0

    lax.fori_loop(0, n // 16, body, 0)


_CH2S = _P2P // 16
_NC2S = _CH2S // _W2


def _k0_body(cm_hbm, code_hbm, zcode, zcm):
    wid = lax.axis_index("s") * 2 + lax.axis_index("c")
    _zero_fill(zcode, _ZB)
    _zero_fill(zcm, _ZBC)
    cbase = wid * (4 * _ZB)
    for k in range(4):
        pltpu.sync_copy(zcode, code_hbm.at[pl.ds(cbase + k * _ZB, _ZB)])
    cmbase = wid * (8 * _ZBC)
    for k in range(8):
        pltpu.sync_copy(zcm, cm_hbm.at[pl.ds(cmbase + k * _ZBC, _ZBC)])


def _build_slot_sh(p2_hbm, slot_sh, zneg, p2f, p2v, jvals, sem, sid):
    negone = jnp.full((16,), -1, jnp.int32)

    def nbody(i, _):
        zneg[pl.ds(i * 16, 16)] = negone
        return 0

    lax.fori_loop(0, _ZB // 16, nbody, 0)
    sbase = sid * (8 * _ZB)
    for k in range(8):
        pltpu.sync_copy(zneg, slot_sh.at[pl.ds(sbase + k * _ZB, _ZB)])
    plsc.subcore_barrier()
    base2 = sid * _CH2S
    _load_2d(p2_hbm, base2, p2f, p2v, _NC2S, _W2, sem)
    iota = lax.iota(jnp.int32, 16)

    def jbody(i, _):
        jvals[pl.ds(i * 16, 16)] = base2 + i * 16 + iota
        return 0

    lax.fori_loop(0, _CH2S // 16, jbody, 0)
    for cc in range(_NC2S):
        pltpu.sync_copy(jvals.at[pl.ds(cc * _W2, _W2)], slot_sh.at[p2v.at[cc]])
    plsc.subcore_barrier()


def _load_2d(hbm, base, flat, ref2d, nc, w, sem):
    pltpu.sync_copy(hbm.at[pl.ds(base, nc * w)], flat)
    for cc in range(nc):
        def body(k, _, cc=cc):
            ref2d[cc, pl.ds(k * 16, 16)] = flat[pl.ds(cc * w + k * 16, 16)]
            return 0

        lax.fori_loop(0, w // 16, body, 0)


def _k12_body(p2_hbm, p1_hbm, v1_hbm, cm_hbm, dummy_hbm,
              slot_sh, zneg, p2f, p2v, jvals, p1v, v1v, jjv, cmi, ones, sem):
    cid = lax.axis_index("c")
    sid = lax.axis_index("s")
    _build_slot_sh(p2_hbm, slot_sh, zneg, p2f, p2v, jvals, sem, sid)
    onev = jnp.full((16,), 1.0, jnp.float32)

    def obody(i, _):
        ones[pl.ds(i * 16, 16)] = onev
        return 0

    lax.fori_loop(0, _W1 // 16, obody, 0)
    base = cid * (_P1P // 2) + sid * _CH1
    pltpu.sync_copy(p1_hbm.at[pl.ds(base, _CH1)], p1v)
    pltpu.sync_copy(v1_hbm.at[pl.ds(base, _CH1)], v1v)
    for cc in range(_NC1):
        pltpu.sync_copy(
            slot_sh.at[p1v.at[pl.ds(cc * _W1, _W1)]],
            jjv.at[pl.ds(cc * _W1, _W1)],
        )

    iota = lax.iota(jnp.int32, 16)
    for cc in range(_NC1):
        def ibody(k, _, cc=cc):
            s = cc * _W1 + k * 16
            jj = jjv[pl.ds(s, 16)]
            v1 = v1v[pl.ds(s, 16)]
            gidx = base + s + iota
            valid = (jj >= 0) & (jj < _M2) & (gidx < _M1)
            jjc = jnp.clip(jj, 0, _M2 - 1)
            dump = _CM_REAL + gidx
            cmi[cc, pl.ds(k * 16, 16)] = jnp.where(valid, v1 * _M2 + jjc, dump)
            return 0

        lax.fori_loop(0, _W1 // 16, ibody, 0)
    scats = [
        pltpu.async_copy(ones, cm_hbm.at[cmi.at[cc]], sem)
        for cc in range(_NC1)
    ]
    for d in scats:
        d.wait()

    @pl.when((cid == 0) & (sid == 0))
    def _():
        pltpu.sync_copy(jjv.at[pl.ds(0, 16)], dummy_hbm)


def _k4_body(p2_hbm, tcode_hbm, code_hbm, dummy_hbm,
             slot_sh, zneg, p2f, p2v, jvals, p2fb, p2vb, jjv, jjcv, tv, sem):
    cid = lax.axis_index("c")
    sid = lax.axis_index("s")
    _build_slot_sh(p2_hbm, slot_sh, zneg, p2f, p2v, jvals, sem, sid)
    base = cid * (_P2P // 2) + sid * _CH2
    _load_2d(p2_hbm, base, p2fb, p2vb, _NC2, _W2, sem)
    for cc in range(_NC2):
        pltpu.sync_copy(slot_sh.at[p2vb.at[cc]], jjv.at[pl.ds(cc * _W2, _W2)])

    def cbody(i, _):
        jj = jjv[pl.ds(i * 16, 16)]
        jjcv[pl.ds(i * 16, 16)] = jnp.clip(jj, 0, _M2 - 1)
        return 0

    lax.fori_loop(0, _CH2 // 16, cbody, 0)
    g2 = [
        pltpu.async_copy(
            tcode_hbm.at[jjcv.at[pl.ds(cc * _W2, _W2)]],
            tv.at[pl.ds(cc * _W2, _W2)],
            sem,
        )
        for cc in range(_NC2)
    ]
    for d in g2:
        d.wait()
    scats = [
        pltpu.async_copy(
            tv.at[pl.ds(cc * _W2, _W2)], code_hbm.at[p2vb.at[cc]], sem
        )
        for cc in range(_NC2)
    ]
    for d in scats:
        d.wait()

    @pl.when((cid == 0) & (sid == 0))
    def _():
        pltpu.sync_copy(jjcv.at[pl.ds(0, 16)], dummy_hbm)


def _k3_tc(cm2, WT, b_col):
    BJ = 12_288
    G = -(-_M2 // BJ)

    def body(cm_ref, wt_ref, b_ref, o_ref):
        cmb = cm_ref[...]
        touched = jnp.max(cmb, axis=0, keepdims=True) > 0.0
        logits = jnp.dot(
            wt_ref[...], cmb, preferred_element_type=jnp.float32
        ) + b_ref[...]
        m = jnp.max(logits, axis=0, keepdims=True)
        iota_c = lax.broadcasted_iota(jnp.int32, (_C, BJ), 0)
        sidx = jnp.min(
            jnp.where(logits == m, iota_c, _C), axis=0, keepdims=True
        )
        tc = jnp.where(touched, sidx + 2, 1)
        o_ref[...] = tc[0]

    return pl.pallas_call(
        body,
        out_shape=jax.ShapeDtypeStruct((_M2,), jnp.int32),
        grid=(G,),
        in_specs=[
            pl.BlockSpec((_C, BJ), lambda i: (0, i)),
            pl.BlockSpec((_C, _C), lambda i: (0, 0)),
            pl.BlockSpec((_C, 1), lambda i: (0, 0)),
        ],
        out_specs=pl.BlockSpec((BJ,), lambda i: (i,)),
    )(cm2, WT, b_col)


def _k5_tc(primaryT, code, W, b_row):
    BN = 32_768
    G = -(-_N // BN)

    def body(pm_ref, code_ref, w_ref, b_ref, o_ref):
        pm = pm_ref[...]
        codeb = code_ref[...][None, :]
        iota_c = lax.broadcasted_iota(jnp.int32, (_C, BN), 0)
        m = jnp.max(pm, axis=0, keepdims=True)
        k_n = jnp.min(
            jnp.where(pm == m, iota_c, _C), axis=0, keepdims=True
        )
        wb = w_ref[...] + b_ref[...]
        wm = jnp.max(wb, axis=1, keepdims=True)
        iota_cc = lax.broadcasted_iota(jnp.int32, (_C, _C), 1)
        ttab = jnp.min(
            jnp.where(wb == wm, iota_cc, _C), axis=1, keepdims=True
        )
        oh = (iota_c == k_n).astype(jnp.int32)
        tval = jnp.sum(oh * ttab, axis=0, keepdims=True)
        w_n = jnp.where(codeb >= 2, codeb - 2, tval)
        onehot = jnp.where(iota_c == w_n, 1.0, 0.0)
        o_ref[...] = jnp.where(codeb >= 1, onehot, pm)

    return pl.pallas_call(
        body,
        out_shape=jax.ShapeDtypeStruct((_C, _N), jnp.float32),
        grid=(G,),
        in_specs=[
            pl.BlockSpec((_C, BN), lambda i: (0, i)),
            pl.BlockSpec((BN,), lambda i: (i,)),
            pl.BlockSpec((_C, _C), lambda i: (0, 0)),
            pl.BlockSpec((1, _C), lambda i: (0, 0)),
        ],
        out_specs=pl.BlockSpec((_C, BN), lambda i: (0, i)),
    )(primaryT, code, W, b_row)


def kernel(primary, W, b, positions1, values1, positions2):
    p1 = positions1.astype(jnp.int32)
    v1 = values1.astype(jnp.int32)
    p2 = positions2.astype(jnp.int32)
    p1p = jnp.concatenate([p1, jnp.zeros((_P1P - _M1,), jnp.int32)])
    v1p = jnp.concatenate([v1, jnp.zeros((_P1P - _M1,), jnp.int32)])
    p2p = jnp.concatenate(
        [p2, _N + jnp.arange(_P2P - _M2, dtype=jnp.int32)]
    )

    mesh = plsc.VectorSubcoreMesh(core_axis_name="c", subcore_axis_name="s")

    k0 = pl.kernel(
        _k0_body,
        out_type=[
            jax.ShapeDtypeStruct((_CM_SZ,), jnp.float32),
            jax.ShapeDtypeStruct((_CODE_SZ,), jnp.int32),
        ],
        mesh=mesh,
        scratch_types=[
            pltpu.VMEM((_ZB,), jnp.int32),
            pltpu.VMEM((_ZBC,), jnp.float32),
        ],
    )
    cm0, code0 = k0()

    k12 = pl.kernel(
        _k12_body,
        out_type=jax.ShapeDtypeStruct((16,), jnp.int32),
        mesh=mesh,
        scratch_types=[
            pltpu.VMEM_SHARED((_SLOT_SZ,), jnp.int32),
            pltpu.VMEM((_ZB,), jnp.int32),
            pltpu.VMEM((_CH2S,), jnp.int32),
            pltpu.VMEM((_NC2S, _W2), jnp.int32),
            pltpu.VMEM((_CH2S,), jnp.int32),
            pltpu.VMEM((_CH1,), jnp.int32),
            pltpu.VMEM((_CH1,), jnp.int32),
            pltpu.VMEM((_CH1,), jnp.int32),
            pltpu.VMEM((_NC1, _W1), jnp.int32),
            pltpu.VMEM((_W1,), jnp.float32),
            pltpu.SemaphoreType.DMA,
        ],
    )
    cm_ref = jax.new_ref(cm0)
    k12(p2p, p1p, v1p, cm_ref)
    cm = cm_ref[...]

    cm2 = cm[:_CM_REAL].reshape(_C, _M2)
    WT = jnp.swapaxes(W, 0, 1)
    b_col = b.reshape(_C, 1)
    tcode = _k3_tc(cm2, WT, b_col)

    k4 = pl.kernel(
        _k4_body,
        out_type=jax.ShapeDtypeStruct((16,), jnp.int32),
        mesh=mesh,
        scratch_types=[
            pltpu.VMEM_SHARED((_SLOT_SZ,), jnp.int32),
            pltpu.VMEM((_ZB,), jnp.int32),
            pltpu.VMEM((_CH2S,), jnp.int32),
            pltpu.VMEM((_NC2S, _W2), jnp.int32),
            pltpu.VMEM((_CH2S,), jnp.int32),
            pltpu.VMEM((_CH2,), jnp.int32),
            pltpu.VMEM((_NC2, _W2), jnp.int32),
            pltpu.VMEM((_CH2,), jnp.int32),
            pltpu.VMEM((_CH2,), jnp.int32),
            pltpu.VMEM((_CH2,), jnp.int32),
            pltpu.SemaphoreType.DMA,
        ],
    )
    code_ref = jax.new_ref(code0)
    k4(p2p, tcode, code_ref)
    code = code_ref[...]

    primaryT = jnp.swapaxes(primary, 0, 1)
    b_row = b.reshape(1, _C)
    outT = _k5_tc(primaryT, code[:_N], W, b_row)
    return jnp.swapaxes(outT, 0, 1)
